# Initial kernel scaffold; baseline (speedup 1.0000x reference)
import functools

import jax
import jax.numpy as jnp
from jax import lax
from jax.experimental import pallas as pl
from jax.experimental.pallas import tpu as pltpu
from jax.experimental.pallas import tpu_sc as plsc

N = 10000
E = 320000
D = 128
NP = 10240
NEG_SLOPE = 0.01
C = 128
NCHUNK = E // C
NS = 16
K = (NCHUNK + NS - 1) // NS
BR = 2048


def _proj_kernel(h_ref, w_ref, a_ref, z_ref, s_ref):
    z = jnp.dot(h_ref[...], w_ref[0].T, preferred_element_type=jnp.float32)
    z_ref[...] = z
    s_ref[0, 0, :] = z @ a_ref[0, 0, :D]
    s_ref[0, 1, :] = z @ a_ref[0, 0, D:]


def _merge_kernel(acc_ref, den_ref, out_ref):
    den = den_ref[...]
    r = jnp.where(den > 0.0, 1.0 / den, 0.0)
    out_ref[...] = 0.5 * (acc_ref[0, :, :] * r[0, :, None]
                          + acc_ref[1, :, :] * r[1, :, None])


def _sc_body(edge_hbm, z_hbm, s_hbm, out_hbm, den_hbm,
             acc_sh, den_sh, tab_a, tab_b, ee_all, src_all, dst_all,
             rows, zbuf, sem):
    c = lax.axis_index("c")
    s = lax.axis_index("s")
    zero16 = jnp.zeros((16,), jnp.float32)

    def zrows(i, _):
        rows[i // 8, pl.ds((i % 8) * 16, 16)] = zero16
        return 0
    lax.fori_loop(0, (C * D) // 16, zrows, 0)

    def zbuf_zero(i, _):
        zbuf[pl.ds(i * 16, 16)] = zero16
        return 0
    lax.fori_loop(0, 640 // 16, zbuf_zero, 0)

    for kk in range(5):
        pltpu.sync_copy(rows, acc_sh.at[pl.ds((s * 5 + kk) * C, C)])
    pltpu.sync_copy(zbuf, den_sh.at[pl.ds(s * 640, 640)])

    pltpu.sync_copy(s_hbm.at[c, 0], tab_a)
    pltpu.sync_copy(s_hbm.at[c, 1], tab_b)
    plsc.subcore_barrier()

    zoff = c * NP

    def phase_a(k, _):
        g = k * NS + s

        @pl.when(g < NCHUNK)
        def _():
            base = g * C
            pltpu.sync_copy(edge_hbm.at[0, pl.ds(base, C)], src_all.at[k])
            pltpu.sync_copy(edge_hbm.at[1, pl.ds(base, C)], dst_all.at[k])
            for j in range(C // 16):
                sv = src_all[k, pl.ds(j * 16, 16)]
                dv = dst_all[k, pl.ds(j * 16, 16)]
                sg = plsc.load_gather(tab_a, [sv])
                dg = plsc.load_gather(tab_b, [dv])
                e = sg + dg
                e = jnp.where(e > 0.0, e, e * NEG_SLOPE)
                ee_all[pl.ds(k * C + j * 16, 16)] = jnp.exp(e)
                src_all[k, pl.ds(j * 16, 16)] = sv + zoff
            pltpu.sync_copy(ee_all.at[pl.ds(k * C, C)],
                            den_sh.at[dst_all.at[k]], add=True)
        return 0

    lax.fori_loop(0, K, phase_a, 0)
    plsc.subcore_barrier()

    pltpu.sync_copy(den_sh, tab_a)

    def phase_b(k, _):
        g = k * NS + s

        @pl.when(g < NCHUNK)
        def _():
            pltpu.async_copy(z_hbm.at[src_all.at[k]], rows, sem).wait()
            for j in range(C // 16):
                dv = dst_all[k, pl.ds(j * 16, 16)]
                dg = plsc.load_gather(tab_a, [dv])
                ee = ee_all[pl.ds(k * C + j * 16, 16)]
                ee_all[pl.ds(k * C + j * 16, 16)] = ee / dg

            def scale(r, _):
                av = plsc.load_gather(
                    ee_all, [jnp.full((16,), k * C, jnp.int32) + r])
                for q in range(D // 16):
                    rows[r, pl.ds(q * 16, 16)] = rows[r, pl.ds(q * 16, 16)] * av
                return 0

            lax.fori_loop(0, C, scale, 0)
            pltpu.sync_copy(rows, acc_sh.at[dst_all.at[k]], add=True)
        return 0

    lax.fori_loop(0, K, phase_b, 0)
    plsc.subcore_barrier()

    pltpu.sync_copy(acc_sh.at[pl.ds(s * 640, 640)],
                    out_hbm.at[c, pl.ds(s * 640, 640)])
    pltpu.sync_copy(den_sh.at[pl.ds(s * 640, 640)],
                    den_hbm.at[c, pl.ds(s * 640, 640)])


_sc_kernel = functools.partial(
    pl.kernel,
    out_type=(jax.ShapeDtypeStruct((2, NP, D), jnp.float32),
              jax.ShapeDtypeStruct((2, NP), jnp.float32)),
    mesh=plsc.VectorSubcoreMesh(core_axis_name="c", subcore_axis_name="s"),
    scratch_types=[
        pltpu.VMEM_SHARED((NP, D), jnp.float32),
        pltpu.VMEM_SHARED((NP,), jnp.float32),
        pltpu.VMEM((NP,), jnp.float32),
        pltpu.VMEM((NP,), jnp.float32),
        pltpu.VMEM((K * C,), jnp.float32),
        pltpu.VMEM((K, C), jnp.int32),
        pltpu.VMEM((K, C), jnp.int32),
        pltpu.VMEM((C, D), jnp.float32),
        pltpu.VMEM((640,), jnp.float32),
        pltpu.SemaphoreType.DMA,
    ],
)(_sc_body)


@jax.jit
def kernel(h, edge_index, W0, a0, W1, a1):
    w = jnp.stack([W0, W1])
    a = jnp.stack([a0[0], a1[0]])[:, None, :]
    nb = NP // BR
    z_all, s_all = pl.pallas_call(
        _proj_kernel,
        grid=(2, nb),
        in_specs=[
            pl.BlockSpec((BR, D), lambda hd, j: (j, 0)),
            pl.BlockSpec((1, D, D), lambda hd, j: (hd, 0, 0)),
            pl.BlockSpec((1, 1, 2 * D), lambda hd, j: (hd, 0, 0)),
        ],
        out_specs=[
            pl.BlockSpec((BR, D), lambda hd, j: (hd * nb + j, 0)),
            pl.BlockSpec((1, 2, BR), lambda hd, j: (hd, 0, j)),
        ],
        out_shape=[
            jax.ShapeDtypeStruct((2 * NP, D), jnp.float32),
            jax.ShapeDtypeStruct((2, 2, NP), jnp.float32),
        ],
    )(h, w, a)

    acc, den = _sc_kernel(edge_index, z_all, s_all)

    out = pl.pallas_call(
        _merge_kernel,
        grid=(NP // BR,),
        in_specs=[
            pl.BlockSpec((2, BR, D), lambda j: (0, j, 0)),
            pl.BlockSpec((2, BR), lambda j: (0, j)),
        ],
        out_specs=pl.BlockSpec((BR, D), lambda j: (j, 0)),
        out_shape=jax.ShapeDtypeStruct((N, D), jnp.float32),
    )(acc, den)
    return out


# speedup vs baseline: 17.0316x; 17.0316x over previous
import functools

import jax
import jax.numpy as jnp
from jax import lax
from jax.experimental import pallas as pl
from jax.experimental.pallas import tpu as pltpu
from jax.experimental.pallas import tpu_sc as plsc

N = 10000
E = 320000
D = 128
NP = 10240
NEG_SLOPE = 0.01
C = 128
NCHUNK = E // C
NS = 16
K = (NCHUNK + NS - 1) // NS
BR = 2048


def _proj_kernel(h_ref, w_ref, a_ref, z_ref, s_ref):
    z = jnp.dot(h_ref[...], w_ref[0].T, preferred_element_type=jnp.float32)
    z_ref[...] = z
    s_ref[0, 0, :] = z @ a_ref[0, 0, :D]
    s_ref[0, 1, :] = z @ a_ref[0, 0, D:]


def _merge_kernel(acc_ref, den_ref, out_ref):
    den = den_ref[...]
    r = jnp.where(den > 0.0, 1.0 / den, 0.0)
    out_ref[...] = 0.5 * (acc_ref[0, :, :] * r[0, :, None]
                          + acc_ref[1, :, :] * r[1, :, None])


def _sc_body(edge_hbm, z_hbm, s_hbm, out_hbm, den_hbm,
             acc_sh, den_sh, tab_a, tab_b, src_c, dst_c, ee_c,
             rows, zbuf, sem):
    c = lax.axis_index("c")
    s = lax.axis_index("s")
    zero16 = jnp.zeros((16,), jnp.float32)

    def zrows(i, _):
        rows[i // 8, pl.ds((i % 8) * 16, 16)] = zero16
        return 0
    lax.fori_loop(0, (C * D) // 16, zrows, 0)

    def zbuf_zero(i, _):
        zbuf[pl.ds(i * 16, 16)] = zero16
        return 0
    lax.fori_loop(0, 640 // 16, zbuf_zero, 0)

    for kk in range(5):
        pltpu.sync_copy(rows, acc_sh.at[pl.ds((s * 5 + kk) * C, C)])
    pltpu.sync_copy(zbuf, den_sh.at[pl.ds(s * 640, 640)])

    pltpu.sync_copy(s_hbm.at[c, 0], tab_a)
    pltpu.sync_copy(s_hbm.at[c, 1], tab_b)
    plsc.subcore_barrier()

    zoff = c * NP

    def chunk(k, _):
        g = k * NS + s

        @pl.when(g < NCHUNK)
        def _():
            base = g * C
            pltpu.sync_copy(edge_hbm.at[0, pl.ds(base, C)], src_c)
            pltpu.sync_copy(edge_hbm.at[1, pl.ds(base, C)], dst_c)
            for j in range(C // 16):
                sl = pl.ds(j * 16, 16)
                src_c[sl] = src_c[sl] + zoff
            cp = pltpu.async_copy(z_hbm.at[src_c], rows, sem)
            for j in range(C // 16):
                sl = pl.ds(j * 16, 16)
                sg = plsc.load_gather(tab_a, [src_c[sl] - zoff])
                dg = plsc.load_gather(tab_b, [dst_c[sl]])
                e = sg + dg
                e = jnp.where(e > 0.0, e, e * NEG_SLOPE)
                ee_c[sl] = jnp.exp(e)
            pltpu.sync_copy(ee_c, den_sh.at[dst_c], add=True)
            cp.wait()

            def scale(r, _):
                av = plsc.load_gather(ee_c, [jnp.full((16,), r, jnp.int32)])
                for q in range(D // 16):
                    rows[r, pl.ds(q * 16, 16)] = rows[r, pl.ds(q * 16, 16)] * av
                return 0

            lax.fori_loop(0, C, scale, 0)
            pltpu.sync_copy(rows, acc_sh.at[dst_c], add=True)
        return 0

    lax.fori_loop(0, K, chunk, 0)
    plsc.subcore_barrier()

    pltpu.sync_copy(acc_sh.at[pl.ds(s * 640, 640)],
                    out_hbm.at[c, pl.ds(s * 640, 640)])
    pltpu.sync_copy(den_sh.at[pl.ds(s * 640, 640)],
                    den_hbm.at[c, pl.ds(s * 640, 640)])


_sc_kernel = functools.partial(
    pl.kernel,
    out_type=(jax.ShapeDtypeStruct((2, NP, D), jnp.float32),
              jax.ShapeDtypeStruct((2, NP), jnp.float32)),
    mesh=plsc.VectorSubcoreMesh(core_axis_name="c", subcore_axis_name="s"),
    compiler_params=pltpu.CompilerParams(needs_layout_passes=False),
    scratch_types=[
        pltpu.VMEM_SHARED((NP, D), jnp.float32),
        pltpu.VMEM_SHARED((NP,), jnp.float32),
        pltpu.VMEM((NP,), jnp.float32),
        pltpu.VMEM((NP,), jnp.float32),
        pltpu.VMEM((C,), jnp.int32),
        pltpu.VMEM((C,), jnp.int32),
        pltpu.VMEM((C,), jnp.float32),
        pltpu.VMEM((C, D), jnp.float32),
        pltpu.VMEM((640,), jnp.float32),
        pltpu.SemaphoreType.DMA,
    ],
)(_sc_body)


@jax.jit
def kernel(h, edge_index, W0, a0, W1, a1):
    w = jnp.stack([W0, W1])
    a = jnp.stack([a0[0], a1[0]])[:, None, :]
    nb = NP // BR
    z_all, s_all = pl.pallas_call(
        _proj_kernel,
        grid=(2, nb),
        in_specs=[
            pl.BlockSpec((BR, D), lambda hd, j: (j, 0)),
            pl.BlockSpec((1, D, D), lambda hd, j: (hd, 0, 0)),
            pl.BlockSpec((1, 1, 2 * D), lambda hd, j: (hd, 0, 0)),
        ],
        out_specs=[
            pl.BlockSpec((BR, D), lambda hd, j: (hd * nb + j, 0)),
            pl.BlockSpec((1, 2, BR), lambda hd, j: (hd, 0, j)),
        ],
        out_shape=[
            jax.ShapeDtypeStruct((2 * NP, D), jnp.float32),
            jax.ShapeDtypeStruct((2, 2, NP), jnp.float32),
        ],
    )(h, w, a)

    acc, den = _sc_kernel(edge_index, z_all, s_all)

    out = pl.pallas_call(
        _merge_kernel,
        grid=(NP // BR,),
        in_specs=[
            pl.BlockSpec((2, BR, D), lambda j: (0, j, 0)),
            pl.BlockSpec((2, BR), lambda j: (0, j)),
        ],
        out_specs=pl.BlockSpec((BR, D), lambda j: (j, 0)),
        out_shape=jax.ShapeDtypeStruct((N, D), jnp.float32),
    )(acc, den)
    return out


# speedup vs baseline: 19.8468x; 1.1653x over previous
import functools

import jax
import jax.numpy as jnp
from jax import lax
from jax.experimental import pallas as pl
from jax.experimental.pallas import tpu as pltpu
from jax.experimental.pallas import tpu_sc as plsc

N = 10000
E = 320000
D = 128
NP = 10240
NEG_SLOPE = 0.01
C = 64
NCHUNK = E // C
NS = 16
NSUB = 324
CPT = 320
B4 = (NS * CPT) // 4
BR = 2048


def _proj_kernel(h_ref, w_ref, a_ref, z_ref, s_ref):
    z = jnp.dot(h_ref[...], w_ref[0].T, preferred_element_type=jnp.float32)
    z_ref[...] = z
    s_ref[0, 0, :] = z @ a_ref[0, 0, :D]
    s_ref[0, 1, :] = z @ a_ref[0, 0, D:]


def _merge_kernel(acc_ref, den_ref, out_ref):
    den = den_ref[...]
    r = jnp.where(den > 0.0, 1.0 / den, 0.0)
    out_ref[...] = 0.5 * (acc_ref[0, :, :] * r[0, :, None]
                          + acc_ref[1, :, :] * r[1, :, None])


def _sc_body(src2d_hbm, dst2d_hbm, z_hbm, s_hbm, out_hbm, den_hbm,
             acc_sh, den_sh, tab_a, tab_b, srcb, dstb, eeb,
             rows, zbuf, sem_g, sem_s0, sem_s1):
    c = lax.axis_index("c")
    s = lax.axis_index("s")
    zero16 = jnp.zeros((16,), jnp.float32)
    ssems = (sem_s0, sem_s1)

    cnt = jnp.clip(NCHUNK - s * CPT, 0, CPT)
    start4 = s * (CPT // 4)

    def zrows(i, _):
        rows[i // 512, (i // 8) % C, pl.ds((i % 8) * 16, 16)] = zero16
        return 0
    lax.fori_loop(0, (2 * C * D) // 16, zrows, 0)

    def zbuf_zero(i, _):
        zbuf[pl.ds(i * 16, 16)] = zero16
        return 0
    lax.fori_loop(0, 640 // 16, zbuf_zero, 0)

    for kk in range(10):
        pltpu.sync_copy(rows.at[0], acc_sh.at[pl.ds((s * 10 + kk) * C, C)])
    pltpu.sync_copy(zbuf, den_sh.at[pl.ds(s * 640, 640)])

    pltpu.sync_copy(s_hbm.at[c, 0], tab_a)
    pltpu.sync_copy(s_hbm.at[c, 1], tab_b)
    plsc.subcore_barrier()

    zoff = c * NP

    def outer(k, _):
        mb = k * 4
        sb = (k % 2) * 4

        @pl.when(mb < cnt)
        def _():
            pltpu.sync_copy(src2d_hbm.at[start4 + k], srcb.at[pl.ds(sb, 4)])
            pltpu.sync_copy(dst2d_hbm.at[start4 + k], dstb.at[pl.ds(sb, 4)])
        for r in range(4):
            m = mb + r
            bm = r % 2
            bj = (r - 1) % 2

            @pl.when(jnp.logical_and(m >= 1, m - 1 < cnt))
            def _():
                sj = (m - 1) % 8
                for q in range(C // 16):
                    sl = pl.ds(q * 16, 16)
                    sv = srcb[sj, sl] - zoff
                    dv = dstb[sj, sl]
                    e = plsc.load_gather(tab_a, [sv]) + plsc.load_gather(tab_b, [dv])
                    e = jnp.where(e > 0.0, e, e * NEG_SLOPE)
                    eeb[sj, sl] = jnp.exp(e)
                pltpu.sync_copy(eeb.at[sj], den_sh.at[dstb.at[sj]], add=True)
                pltpu.make_async_copy(z_hbm.at[srcb.at[sj]],
                                      rows.at[bj], sem_g).wait()

                def scale(r2, _):
                    av = plsc.load_gather(
                        eeb, [jnp.full((16,), sj, jnp.int32),
                              jnp.full((16,), r2, jnp.int32)])
                    for q in range(D // 16):
                        sl = pl.ds(q * 16, 16)
                        rows[bj, r2, sl] = rows[bj, r2, sl] * av
                    return 0

                lax.fori_loop(0, C, scale, 0)
                pltpu.async_copy(rows.at[bj], acc_sh.at[dstb.at[sj]],
                                 ssems[bj], add=True)

            @pl.when(m < cnt)
            def _():
                sm = m % 8

                @pl.when(m >= 2)
                def _():
                    pltpu.make_async_copy(rows.at[bm], acc_sh.at[dstb.at[0]],
                                          ssems[bm]).wait()

                for q in range(C // 16):
                    sl = pl.ds(q * 16, 16)
                    srcb[sm, sl] = srcb[sm, sl] + zoff
                pltpu.async_copy(z_hbm.at[srcb.at[sm]], rows.at[bm], sem_g)
        return 0

    lax.fori_loop(0, NSUB // 4, outer, 0)
    pltpu.make_async_copy(rows.at[0], acc_sh.at[dstb.at[0]], sem_s0).wait()
    pltpu.make_async_copy(rows.at[1], acc_sh.at[dstb.at[0]], sem_s1).wait()
    plsc.subcore_barrier()

    pltpu.sync_copy(acc_sh.at[pl.ds(s * 640, 640)],
                    out_hbm.at[c, pl.ds(s * 640, 640)])
    pltpu.sync_copy(den_sh.at[pl.ds(s * 640, 640)],
                    den_hbm.at[c, pl.ds(s * 640, 640)])


_sc_kernel = functools.partial(
    pl.kernel,
    out_type=(jax.ShapeDtypeStruct((2, NP, D), jnp.float32),
              jax.ShapeDtypeStruct((2, NP), jnp.float32)),
    mesh=plsc.VectorSubcoreMesh(core_axis_name="c", subcore_axis_name="s"),
    compiler_params=pltpu.CompilerParams(needs_layout_passes=False),
    scratch_types=[
        pltpu.VMEM_SHARED((NP, D), jnp.float32),
        pltpu.VMEM_SHARED((NP,), jnp.float32),
        pltpu.VMEM((NP,), jnp.float32),
        pltpu.VMEM((NP,), jnp.float32),
        pltpu.VMEM((8, C), jnp.int32),
        pltpu.VMEM((8, C), jnp.int32),
        pltpu.VMEM((8, C), jnp.float32),
        pltpu.VMEM((2, C, D), jnp.float32),
        pltpu.VMEM((640,), jnp.float32),
        pltpu.SemaphoreType.DMA,
        pltpu.SemaphoreType.DMA,
        pltpu.SemaphoreType.DMA,
    ],
)(_sc_body)


@jax.jit
def kernel(h, edge_index, W0, a0, W1, a1):
    w = jnp.stack([W0, W1])
    a = jnp.stack([a0[0], a1[0]])[:, None, :]
    nb = NP // BR
    z_all, s_all = pl.pallas_call(
        _proj_kernel,
        grid=(2, nb),
        in_specs=[
            pl.BlockSpec((BR, D), lambda hd, j: (j, 0)),
            pl.BlockSpec((1, D, D), lambda hd, j: (hd, 0, 0)),
            pl.BlockSpec((1, 1, 2 * D), lambda hd, j: (hd, 0, 0)),
        ],
        out_specs=[
            pl.BlockSpec((BR, D), lambda hd, j: (hd * nb + j, 0)),
            pl.BlockSpec((1, 2, BR), lambda hd, j: (hd, 0, j)),
        ],
        out_shape=[
            jax.ShapeDtypeStruct((2 * NP, D), jnp.float32),
            jax.ShapeDtypeStruct((2, 2, NP), jnp.float32),
        ],
    )(h, w, a)

    ei = edge_index.reshape(2, NCHUNK, C)
    ei = jnp.concatenate(
        [ei, jnp.zeros((2, NS * CPT - NCHUNK, C), jnp.int32)], axis=1)
    ei = ei.reshape(2, B4, 4, C)
    acc, den = _sc_kernel(ei[0], ei[1], z_all, s_all)

    out = pl.pallas_call(
        _merge_kernel,
        grid=(NP // BR,),
        in_specs=[
            pl.BlockSpec((2, BR, D), lambda j: (0, j, 0)),
            pl.BlockSpec((2, BR), lambda j: (0, j)),
        ],
        out_specs=pl.BlockSpec((BR, D), lambda j: (j, 0)),
        out_shape=jax.ShapeDtypeStruct((N, D), jnp.float32),
    )(acc, den)
    return out


# speedup vs baseline: 29.7732x; 1.5002x over previous
import functools

import jax
import jax.numpy as jnp
from jax import lax
from jax.experimental import pallas as pl
from jax.experimental.pallas import tpu as pltpu
from jax.experimental.pallas import tpu_sc as plsc

N = 10000
E = 320000
D = 128
NP = 10240
NEG_SLOPE = 0.01
C = 64
NCHUNK = E // C
NS = 16
NSUB = 324
CPT = 320
B4 = (NS * CPT) // 4
BR = 2048


def _proj_kernel(h_ref, w_ref, a_ref, z_ref, s_ref):
    z = jnp.dot(h_ref[...], w_ref[0].T, preferred_element_type=jnp.float32)
    z_ref[...] = z
    s_ref[0, 0, :] = z @ a_ref[0, 0, :D]
    s_ref[0, 1, :] = z @ a_ref[0, 0, D:]


def _merge_kernel(acc_ref, den_ref, out_ref):
    den = den_ref[...]
    r = jnp.where(den > 0.0, 1.0 / den, 0.0)
    out_ref[...] = 0.5 * (acc_ref[0, :, :] * r[0, :, None]
                          + acc_ref[1, :, :] * r[1, :, None])


def _sc_body(src2d_hbm, dst2d_hbm, z_hbm, s_hbm, out_hbm, den_hbm,
             acc_sh, den_sh, tab_a, tab_b, srcb, dstb, eeb,
             rows, zbuf, sem_g0, sem_g1, sem_s0, sem_s1,
             sem_e0, sem_d0, sem_d1):
    c = lax.axis_index("c")
    s = lax.axis_index("s")
    zero16 = jnp.zeros((16,), jnp.float32)
    gsems = (sem_g0, sem_g1)
    ssems = (sem_s0, sem_s1)
    dsems = (sem_d0, sem_d1)

    cnt = jnp.clip(NCHUNK - s * CPT, 0, CPT)
    start4 = s * (CPT // 4)

    def zrows(i, _):
        rows[i // 512, (i // 8) % C, pl.ds((i % 8) * 16, 16)] = zero16
        return 0
    lax.fori_loop(0, (2 * C * D) // 16, zrows, 0)

    def zbuf_zero(i, _):
        zbuf[pl.ds(i * 16, 16)] = zero16
        return 0
    lax.fori_loop(0, 640 // 16, zbuf_zero, 0)

    for kk in range(10):
        pltpu.sync_copy(rows.at[0], acc_sh.at[pl.ds((s * 10 + kk) * C, C)])
    pltpu.sync_copy(zbuf, den_sh.at[pl.ds(s * 640, 640)])

    pltpu.async_copy(src2d_hbm.at[start4], srcb.at[pl.ds(0, 4)], sem_e0)
    pltpu.async_copy(dst2d_hbm.at[start4], dstb.at[pl.ds(0, 4)], sem_e0)

    pltpu.sync_copy(s_hbm.at[c, 0], tab_a)
    pltpu.sync_copy(s_hbm.at[c, 1], tab_b)
    plsc.subcore_barrier()

    zoff = c * NP

    def outer(kk, _):
      for b in range(3):
        k = kk * 3 + b
        mb = k * 4
        g0 = (b % 3) * 4
        gp = ((b + 2) % 3) * 4
        gn = ((b + 1) % 3) * 4

        @pl.when(mb < cnt)
        def _():
            pltpu.make_async_copy(src2d_hbm.at[start4 + k],
                                  srcb.at[pl.ds(g0, 4)], sem_e0).wait()
            pltpu.make_async_copy(dst2d_hbm.at[start4 + k],
                                  dstb.at[pl.ds(g0, 4)], sem_e0).wait()

        @pl.when(mb + 4 < cnt)
        def _():
            pltpu.async_copy(src2d_hbm.at[start4 + k + 1],
                             srcb.at[pl.ds(gn, 4)], sem_e0)
            pltpu.async_copy(dst2d_hbm.at[start4 + k + 1],
                             dstb.at[pl.ds(gn, 4)], sem_e0)

        for r in range(4):
            m = mb + r
            bm = r % 2
            bj = (r - 1) % 2
            se_m = g0 + r
            se_j = gp + 3 if r == 0 else g0 + (r - 1)

            @pl.when(jnp.logical_and(m >= 1, m - 1 < cnt))
            def _():
                sj = (m - 1) % 8

                @pl.when(m - 1 >= 2)
                def _():
                    pltpu.make_async_copy(eeb.at[0], den_sh.at[dstb.at[0]],
                                          dsems[bj]).wait()

                for q in range(C // 16):
                    sl = pl.ds(q * 16, 16)
                    sv = srcb[se_j, sl] - zoff
                    dv = dstb[se_j, sl]
                    e = plsc.load_gather(tab_a, [sv]) + plsc.load_gather(tab_b, [dv])
                    e = jnp.where(e > 0.0, e, e * NEG_SLOPE)
                    eeb[sj, sl] = jnp.exp(e)
                pltpu.async_copy(eeb.at[sj], den_sh.at[dstb.at[se_j]],
                                 dsems[bj], add=True)

            @pl.when(m < cnt)
            def _():
                @pl.when(m >= 2)
                def _():
                    pltpu.make_async_copy(rows.at[bm], acc_sh.at[dstb.at[0]],
                                          ssems[bm]).wait()

                for q in range(C // 16):
                    sl = pl.ds(q * 16, 16)
                    srcb[se_m, sl] = srcb[se_m, sl] + zoff
                pltpu.async_copy(z_hbm.at[srcb.at[se_m]], rows.at[bm],
                                 gsems[bm])

            @pl.when(jnp.logical_and(m >= 1, m - 1 < cnt))
            def _():
                sj = (m - 1) % 8
                pltpu.make_async_copy(z_hbm.at[srcb.at[se_j]],
                                      rows.at[bj], gsems[bj]).wait()

                def scale(r2, _):
                    av = plsc.load_gather(
                        eeb, [jnp.full((16,), sj, jnp.int32),
                              jnp.full((16,), r2, jnp.int32)])
                    for q in range(D // 16):
                        sl = pl.ds(q * 16, 16)
                        rows[bj, r2, sl] = rows[bj, r2, sl] * av
                    return 0

                lax.fori_loop(0, C, scale, 0)
                pltpu.async_copy(rows.at[bj], acc_sh.at[dstb.at[se_j]],
                                 ssems[bj], add=True)
      return 0

    lax.fori_loop(0, NSUB // 12, outer, 0)
    pltpu.make_async_copy(rows.at[0], acc_sh.at[dstb.at[0]], sem_s0).wait()
    pltpu.make_async_copy(rows.at[1], acc_sh.at[dstb.at[0]], sem_s1).wait()
    pltpu.make_async_copy(eeb.at[0], den_sh.at[dstb.at[0]], sem_d0).wait()
    pltpu.make_async_copy(eeb.at[0], den_sh.at[dstb.at[0]], sem_d1).wait()
    plsc.subcore_barrier()

    pltpu.sync_copy(acc_sh.at[pl.ds(s * 640, 640)],
                    out_hbm.at[c, pl.ds(s * 640, 640)])
    pltpu.sync_copy(den_sh.at[pl.ds(s * 640, 640)],
                    den_hbm.at[c, pl.ds(s * 640, 640)])


_sc_kernel = functools.partial(
    pl.kernel,
    out_type=(jax.ShapeDtypeStruct((2, NP, D), jnp.float32),
              jax.ShapeDtypeStruct((2, NP), jnp.float32)),
    mesh=plsc.VectorSubcoreMesh(core_axis_name="c", subcore_axis_name="s"),
    compiler_params=pltpu.CompilerParams(needs_layout_passes=False),
    scratch_types=[
        pltpu.VMEM_SHARED((NP, D), jnp.float32),
        pltpu.VMEM_SHARED((NP,), jnp.float32),
        pltpu.VMEM((NP,), jnp.float32),
        pltpu.VMEM((NP,), jnp.float32),
        pltpu.VMEM((12, C), jnp.int32),
        pltpu.VMEM((12, C), jnp.int32),
        pltpu.VMEM((8, C), jnp.float32),
        pltpu.VMEM((2, C, D), jnp.float32),
        pltpu.VMEM((640,), jnp.float32),
        pltpu.SemaphoreType.DMA,
        pltpu.SemaphoreType.DMA,
        pltpu.SemaphoreType.DMA,
        pltpu.SemaphoreType.DMA,
        pltpu.SemaphoreType.DMA,
        pltpu.SemaphoreType.DMA,
        pltpu.SemaphoreType.DMA,
    ],
)(_sc_body)


@jax.jit
def kernel(h, edge_index, W0, a0, W1, a1):
    w = jnp.stack([W0, W1])
    a = jnp.stack([a0[0], a1[0]])[:, None, :]
    nb = NP // BR
    z_all, s_all = pl.pallas_call(
        _proj_kernel,
        grid=(2, nb),
        in_specs=[
            pl.BlockSpec((BR, D), lambda hd, j: (j, 0)),
            pl.BlockSpec((1, D, D), lambda hd, j: (hd, 0, 0)),
            pl.BlockSpec((1, 1, 2 * D), lambda hd, j: (hd, 0, 0)),
        ],
        out_specs=[
            pl.BlockSpec((BR, D), lambda hd, j: (hd * nb + j, 0)),
            pl.BlockSpec((1, 2, BR), lambda hd, j: (hd, 0, j)),
        ],
        out_shape=[
            jax.ShapeDtypeStruct((2 * NP, D), jnp.float32),
            jax.ShapeDtypeStruct((2, 2, NP), jnp.float32),
        ],
    )(h, w, a)

    ei = edge_index.reshape(2, NCHUNK, C)
    ei = jnp.concatenate(
        [ei, jnp.zeros((2, NS * CPT - NCHUNK, C), jnp.int32)], axis=1)
    ei = ei.reshape(2, B4, 4, C)
    acc, den = _sc_kernel(ei[0], ei[1], z_all, s_all)

    out = pl.pallas_call(
        _merge_kernel,
        grid=(NP // BR,),
        in_specs=[
            pl.BlockSpec((2, BR, D), lambda j: (0, j, 0)),
            pl.BlockSpec((2, BR), lambda j: (0, j)),
        ],
        out_specs=pl.BlockSpec((BR, D), lambda j: (j, 0)),
        out_shape=jax.ShapeDtypeStruct((N, D), jnp.float32),
    )(acc, den)
    return out


# speedup vs baseline: 30.2782x; 1.0170x over previous
import functools

import jax
import jax.numpy as jnp
from jax import lax
from jax.experimental import pallas as pl
from jax.experimental.pallas import tpu as pltpu
from jax.experimental.pallas import tpu_sc as plsc

N = 10000
E = 320000
D = 128
NP = 10240
NEG_SLOPE = 0.01
C = 64
NCHUNK = E // C
NS = 16
NSUB = 324
CPT = 320
B4 = NCHUNK // 4
BR = 2048


def _proj_kernel(h_ref, w_ref, a_ref, z_ref, s_ref):
    z = jnp.dot(h_ref[...], w_ref[0].T, preferred_element_type=jnp.float32)
    z_ref[0, :, :] = z
    s_ref[0, 0, :] = z @ a_ref[0, 0, :D]
    s_ref[0, 1, :] = z @ a_ref[0, 0, D:]


def _merge_kernel(acc_ref, den_ref, out_ref):
    den = den_ref[...]
    r = jnp.where(den > 0.0, 1.0 / den, 0.0)
    out_ref[...] = 0.5 * (acc_ref[0, :, :] * r[0, :, None]
                          + acc_ref[1, :, :] * r[1, :, None])


def _sc_body(src2d_hbm, dst2d_hbm, z_hbm, s_hbm, out_hbm, den_hbm,
             acc_sh, den_sh, tab_a, tab_b, srcb, dstb, eeb,
             rows, zbuf, sem_g0, sem_g1, sem_s0, sem_s1,
             sem_e0, sem_d0, sem_d1):
    c = lax.axis_index("c")
    s = lax.axis_index("s")
    zero16 = jnp.zeros((16,), jnp.float32)
    gsems = (sem_g0, sem_g1)
    ssems = (sem_s0, sem_s1)
    dsems = (sem_d0, sem_d1)

    cnt = jnp.clip(NCHUNK - s * CPT, 0, CPT)
    start4 = s * (CPT // 4)

    def zrows(i, _):
        rows[i // 512, (i // 8) % C, pl.ds((i % 8) * 16, 16)] = zero16
        return 0
    lax.fori_loop(0, (2 * C * D) // 16, zrows, 0)

    def zbuf_zero(i, _):
        zbuf[pl.ds(i * 16, 16)] = zero16
        return 0
    lax.fori_loop(0, 640 // 16, zbuf_zero, 0)

    for kk in range(10):
        pltpu.sync_copy(rows.at[0], acc_sh.at[pl.ds((s * 10 + kk) * C, C)])
    pltpu.sync_copy(zbuf, den_sh.at[pl.ds(s * 640, 640)])

    pltpu.async_copy(src2d_hbm.at[start4], srcb.at[pl.ds(0, 4)], sem_e0)
    pltpu.async_copy(dst2d_hbm.at[start4], dstb.at[pl.ds(0, 4)], sem_e0)

    pltpu.sync_copy(s_hbm.at[c, 0], tab_a)
    pltpu.sync_copy(s_hbm.at[c, 1], tab_b)
    plsc.subcore_barrier()

    def outer(kk, _):
      for b in range(3):
        k = kk * 3 + b
        mb = k * 4
        g0 = (b % 3) * 4
        gp = ((b + 2) % 3) * 4
        gn = ((b + 1) % 3) * 4

        @pl.when(mb < cnt)
        def _():
            pltpu.make_async_copy(src2d_hbm.at[start4 + k],
                                  srcb.at[pl.ds(g0, 4)], sem_e0).wait()
            pltpu.make_async_copy(dst2d_hbm.at[start4 + k],
                                  dstb.at[pl.ds(g0, 4)], sem_e0).wait()

        @pl.when(mb + 4 < cnt)
        def _():
            pltpu.async_copy(src2d_hbm.at[start4 + k + 1],
                             srcb.at[pl.ds(gn, 4)], sem_e0)
            pltpu.async_copy(dst2d_hbm.at[start4 + k + 1],
                             dstb.at[pl.ds(gn, 4)], sem_e0)

        for r in range(4):
            m = mb + r
            bm = r % 2
            bj = (r - 1) % 2
            se_m = g0 + r
            se_j = gp + 3 if r == 0 else g0 + (r - 1)

            @pl.when(jnp.logical_and(m >= 1, m - 1 < cnt))
            def _():
                sj = (m - 1) % 8

                @pl.when(m - 1 >= 2)
                def _():
                    pltpu.make_async_copy(eeb.at[0], den_sh.at[dstb.at[0]],
                                          dsems[bj]).wait()

                for q in range(C // 16):
                    sl = pl.ds(q * 16, 16)
                    sv = srcb[se_j, sl]
                    dv = dstb[se_j, sl]
                    e = plsc.load_gather(tab_a, [sv]) + plsc.load_gather(tab_b, [dv])
                    e = jnp.where(e > 0.0, e, e * NEG_SLOPE)
                    eeb[sj, sl] = jnp.exp(e)
                pltpu.async_copy(eeb.at[sj], den_sh.at[dstb.at[se_j]],
                                 dsems[bj], add=True)

            @pl.when(m < cnt)
            def _():
                @pl.when(m >= 2)
                def _():
                    pltpu.make_async_copy(rows.at[bm], acc_sh.at[dstb.at[0]],
                                          ssems[bm]).wait()

                pltpu.async_copy(z_hbm.at[c].at[srcb.at[se_m]], rows.at[bm],
                                 gsems[bm])

            @pl.when(jnp.logical_and(m >= 1, m - 1 < cnt))
            def _():
                sj = (m - 1) % 8
                pltpu.make_async_copy(z_hbm.at[c].at[srcb.at[se_j]],
                                      rows.at[bj], gsems[bj]).wait()

                def scale(g, _):
                    eev = eeb[sj, pl.ds(g * 16, 16)]
                    for u in range(16):
                        av = eev[u]
                        rr = g * 16 + u
                        for q in range(D // 16):
                            sl = pl.ds(q * 16, 16)
                            rows[bj, rr, sl] = rows[bj, rr, sl] * av
                    return 0

                lax.fori_loop(0, C // 16, scale, 0)
                pltpu.async_copy(rows.at[bj], acc_sh.at[dstb.at[se_j]],
                                 ssems[bj], add=True)
      return 0

    lax.fori_loop(0, NSUB // 12, outer, 0)
    pltpu.make_async_copy(rows.at[0], acc_sh.at[dstb.at[0]], sem_s0).wait()
    pltpu.make_async_copy(rows.at[1], acc_sh.at[dstb.at[0]], sem_s1).wait()
    pltpu.make_async_copy(eeb.at[0], den_sh.at[dstb.at[0]], sem_d0).wait()
    pltpu.make_async_copy(eeb.at[0], den_sh.at[dstb.at[0]], sem_d1).wait()
    plsc.subcore_barrier()

    pltpu.sync_copy(acc_sh.at[pl.ds(s * 640, 640)],
                    out_hbm.at[c, pl.ds(s * 640, 640)])
    pltpu.sync_copy(den_sh.at[pl.ds(s * 640, 640)],
                    den_hbm.at[c, pl.ds(s * 640, 640)])


_sc_kernel = functools.partial(
    pl.kernel,
    out_type=(jax.ShapeDtypeStruct((2, NP, D), jnp.float32),
              jax.ShapeDtypeStruct((2, NP), jnp.float32)),
    mesh=plsc.VectorSubcoreMesh(core_axis_name="c", subcore_axis_name="s"),
    compiler_params=pltpu.CompilerParams(needs_layout_passes=False),
    scratch_types=[
        pltpu.VMEM_SHARED((NP, D), jnp.float32),
        pltpu.VMEM_SHARED((NP,), jnp.float32),
        pltpu.VMEM((NP,), jnp.float32),
        pltpu.VMEM((NP,), jnp.float32),
        pltpu.VMEM((12, C), jnp.int32),
        pltpu.VMEM((12, C), jnp.int32),
        pltpu.VMEM((8, C), jnp.float32),
        pltpu.VMEM((2, C, D), jnp.float32),
        pltpu.VMEM((640,), jnp.float32),
        pltpu.SemaphoreType.DMA,
        pltpu.SemaphoreType.DMA,
        pltpu.SemaphoreType.DMA,
        pltpu.SemaphoreType.DMA,
        pltpu.SemaphoreType.DMA,
        pltpu.SemaphoreType.DMA,
        pltpu.SemaphoreType.DMA,
    ],
)(_sc_body)


@jax.jit
def kernel(h, edge_index, W0, a0, W1, a1):
    w = jnp.stack([W0, W1])
    a = jnp.stack([a0[0], a1[0]])[:, None, :]
    nb = NP // BR
    z_all, s_all = pl.pallas_call(
        _proj_kernel,
        grid=(2, nb),
        in_specs=[
            pl.BlockSpec((BR, D), lambda hd, j: (j, 0)),
            pl.BlockSpec((1, D, D), lambda hd, j: (hd, 0, 0)),
            pl.BlockSpec((1, 1, 2 * D), lambda hd, j: (hd, 0, 0)),
        ],
        out_specs=[
            pl.BlockSpec((1, BR, D), lambda hd, j: (hd, j, 0)),
            pl.BlockSpec((1, 2, BR), lambda hd, j: (hd, 0, j)),
        ],
        out_shape=[
            jax.ShapeDtypeStruct((2, NP, D), jnp.float32),
            jax.ShapeDtypeStruct((2, 2, NP), jnp.float32),
        ],
    )(h, w, a)

    ei = edge_index.reshape(2, B4, 4, C)
    acc, den = _sc_kernel(ei[0], ei[1], z_all, s_all)

    out = pl.pallas_call(
        _merge_kernel,
        grid=(NP // BR,),
        in_specs=[
            pl.BlockSpec((2, BR, D), lambda j: (0, j, 0)),
            pl.BlockSpec((2, BR), lambda j: (0, j)),
        ],
        out_specs=pl.BlockSpec((BR, D), lambda j: (j, 0)),
        out_shape=jax.ShapeDtypeStruct((N, D), jnp.float32),
    )(acc, den)
    return out


# speedup vs baseline: 30.9839x; 1.0233x over previous
import functools

import jax
import jax.numpy as jnp
from jax import lax
from jax.experimental import pallas as pl
from jax.experimental.pallas import tpu as pltpu
from jax.experimental.pallas import tpu_sc as plsc

N = 10000
E = 320000
D = 128
NP = 10240
NEG_SLOPE = 0.01
C = 64
NCHUNK = E // C
NS = 16
NSUB = 324
B4 = NCHUNK // 4
BR = 2048


def _proj_kernel(h_ref, w_ref, a_ref, z_ref, s_ref):
    z = jnp.dot(h_ref[...], w_ref[0].T, preferred_element_type=jnp.float32)
    z_ref[0, :, :] = z
    s_ref[0, 0, :] = z @ a_ref[0, 0, :D]
    s_ref[0, 1, :] = z @ a_ref[0, 0, D:]


def _merge_kernel(acc_ref, den_ref, out_ref):
    den = den_ref[...]
    r = jnp.where(den > 0.0, 1.0 / den, 0.0)
    out_ref[...] = 0.5 * (acc_ref[0, :, :] * r[0, :, None]
                          + acc_ref[1, :, :] * r[1, :, None])


def _sc_body(src2d_hbm, dst2d_hbm, z_hbm, s_hbm, out_hbm, den_hbm,
             acc_sh, den_sh, tab_a, tab_b, srcb, dstb, eeb,
             rows, zbuf, sem_g0, sem_g1, sem_s0, sem_s1,
             sem_e0, sem_d0, sem_d1):
    c = lax.axis_index("c")
    s = lax.axis_index("s")
    zero16 = jnp.zeros((16,), jnp.float32)
    gsems = (sem_g0, sem_g1)
    ssems = (sem_s0, sem_s1)
    dsems = (sem_d0, sem_d1)

    cnt = jnp.where(s < 2, 316, 312)
    start4 = s * 78 + jnp.minimum(s, 2)

    def zrows(i, _):
        rows[i // 512, (i // 8) % C, pl.ds((i % 8) * 16, 16)] = zero16
        return 0
    lax.fori_loop(0, (2 * C * D) // 16, zrows, 0)

    def zbuf_zero(i, _):
        zbuf[pl.ds(i * 16, 16)] = zero16
        return 0
    lax.fori_loop(0, 640 // 16, zbuf_zero, 0)

    for kk in range(10):
        pltpu.sync_copy(rows.at[0], acc_sh.at[pl.ds((s * 10 + kk) * C, C)])
    pltpu.sync_copy(zbuf, den_sh.at[pl.ds(s * 640, 640)])

    pltpu.async_copy(src2d_hbm.at[start4], srcb.at[pl.ds(0, 4)], sem_e0)
    pltpu.async_copy(dst2d_hbm.at[start4], dstb.at[pl.ds(0, 4)], sem_e0)

    pltpu.sync_copy(s_hbm.at[c, 0], tab_a)
    pltpu.sync_copy(s_hbm.at[c, 1], tab_b)
    plsc.subcore_barrier()

    def outer(kk, _):
      for b in range(3):
        k = kk * 3 + b
        mb = k * 4
        g0 = (b % 3) * 4
        gp = ((b + 2) % 3) * 4
        gn = ((b + 1) % 3) * 4

        @pl.when(mb < cnt)
        def _():
            pltpu.make_async_copy(src2d_hbm.at[start4 + k],
                                  srcb.at[pl.ds(g0, 4)], sem_e0).wait()
            pltpu.make_async_copy(dst2d_hbm.at[start4 + k],
                                  dstb.at[pl.ds(g0, 4)], sem_e0).wait()

        @pl.when(mb + 4 < cnt)
        def _():
            pltpu.async_copy(src2d_hbm.at[start4 + k + 1],
                             srcb.at[pl.ds(gn, 4)], sem_e0)
            pltpu.async_copy(dst2d_hbm.at[start4 + k + 1],
                             dstb.at[pl.ds(gn, 4)], sem_e0)

        for r in range(4):
            m = mb + r
            bm = r % 2
            bj = (r - 1) % 2
            se_m = g0 + r
            se_j = gp + 3 if r == 0 else g0 + (r - 1)

            @pl.when(jnp.logical_and(m >= 1, m - 1 < cnt))
            def _():
                sj = (m - 1) % 8

                @pl.when(m - 1 >= 2)
                def _():
                    pltpu.make_async_copy(eeb.at[0], den_sh.at[dstb.at[0]],
                                          dsems[bj]).wait()

                for q in range(C // 16):
                    sl = pl.ds(q * 16, 16)
                    sv = srcb[se_j, sl]
                    dv = dstb[se_j, sl]
                    e = plsc.load_gather(tab_a, [sv]) + plsc.load_gather(tab_b, [dv])
                    e = jnp.where(e > 0.0, e, e * NEG_SLOPE)
                    eeb[sj, sl] = jnp.exp(e)
                pltpu.async_copy(eeb.at[sj], den_sh.at[dstb.at[se_j]],
                                 dsems[bj], add=True)

            @pl.when(m < cnt)
            def _():
                @pl.when(m >= 2)
                def _():
                    pltpu.make_async_copy(rows.at[bm], acc_sh.at[dstb.at[0]],
                                          ssems[bm]).wait()

                pltpu.async_copy(z_hbm.at[c].at[srcb.at[se_m]], rows.at[bm],
                                 gsems[bm])

            @pl.when(jnp.logical_and(m >= 1, m - 1 < cnt))
            def _():
                sj = (m - 1) % 8
                pltpu.make_async_copy(z_hbm.at[c].at[srcb.at[se_j]],
                                      rows.at[bj], gsems[bj]).wait()

                def scale(g, _):
                    eev = eeb[sj, pl.ds(g * 16, 16)]
                    for u in range(16):
                        av = eev[u]
                        rr = g * 16 + u
                        for q in range(D // 16):
                            sl = pl.ds(q * 16, 16)
                            rows[bj, rr, sl] = rows[bj, rr, sl] * av
                    return 0

                lax.fori_loop(0, C // 16, scale, 0)
                pltpu.async_copy(rows.at[bj], acc_sh.at[dstb.at[se_j]],
                                 ssems[bj], add=True)
      return 0

    lax.fori_loop(0, NSUB // 12, outer, 0)
    pltpu.make_async_copy(rows.at[0], acc_sh.at[dstb.at[0]], sem_s0).wait()
    pltpu.make_async_copy(rows.at[1], acc_sh.at[dstb.at[0]], sem_s1).wait()
    pltpu.make_async_copy(eeb.at[0], den_sh.at[dstb.at[0]], sem_d0).wait()
    pltpu.make_async_copy(eeb.at[0], den_sh.at[dstb.at[0]], sem_d1).wait()
    plsc.subcore_barrier()

    pltpu.sync_copy(acc_sh.at[pl.ds(s * 640, 640)],
                    out_hbm.at[c, pl.ds(s * 640, 640)])
    pltpu.sync_copy(den_sh.at[pl.ds(s * 640, 640)],
                    den_hbm.at[c, pl.ds(s * 640, 640)])


_sc_kernel = functools.partial(
    pl.kernel,
    out_type=(jax.ShapeDtypeStruct((2, NP, D), jnp.float32),
              jax.ShapeDtypeStruct((2, NP), jnp.float32)),
    mesh=plsc.VectorSubcoreMesh(core_axis_name="c", subcore_axis_name="s"),
    compiler_params=pltpu.CompilerParams(needs_layout_passes=False),
    scratch_types=[
        pltpu.VMEM_SHARED((NP, D), jnp.float32),
        pltpu.VMEM_SHARED((NP,), jnp.float32),
        pltpu.VMEM((NP,), jnp.float32),
        pltpu.VMEM((NP,), jnp.float32),
        pltpu.VMEM((12, C), jnp.int32),
        pltpu.VMEM((12, C), jnp.int32),
        pltpu.VMEM((8, C), jnp.float32),
        pltpu.VMEM((2, C, D), jnp.float32),
        pltpu.VMEM((640,), jnp.float32),
        pltpu.SemaphoreType.DMA,
        pltpu.SemaphoreType.DMA,
        pltpu.SemaphoreType.DMA,
        pltpu.SemaphoreType.DMA,
        pltpu.SemaphoreType.DMA,
        pltpu.SemaphoreType.DMA,
        pltpu.SemaphoreType.DMA,
    ],
)(_sc_body)


@jax.jit
def kernel(h, edge_index, W0, a0, W1, a1):
    w = jnp.stack([W0, W1])
    a = jnp.stack([a0[0], a1[0]])[:, None, :]
    nb = NP // BR
    z_all, s_all = pl.pallas_call(
        _proj_kernel,
        grid=(2, nb),
        in_specs=[
            pl.BlockSpec((BR, D), lambda hd, j: (j, 0)),
            pl.BlockSpec((1, D, D), lambda hd, j: (hd, 0, 0)),
            pl.BlockSpec((1, 1, 2 * D), lambda hd, j: (hd, 0, 0)),
        ],
        out_specs=[
            pl.BlockSpec((1, BR, D), lambda hd, j: (hd, j, 0)),
            pl.BlockSpec((1, 2, BR), lambda hd, j: (hd, 0, j)),
        ],
        out_shape=[
            jax.ShapeDtypeStruct((2, NP, D), jnp.float32),
            jax.ShapeDtypeStruct((2, 2, NP), jnp.float32),
        ],
    )(h, w, a)

    ei = edge_index.reshape(2, B4, 4, C)
    acc, den = _sc_kernel(ei[0], ei[1], z_all, s_all)

    out = pl.pallas_call(
        _merge_kernel,
        grid=(NP // BR,),
        in_specs=[
            pl.BlockSpec((2, BR, D), lambda j: (0, j, 0)),
            pl.BlockSpec((2, BR), lambda j: (0, j)),
        ],
        out_specs=pl.BlockSpec((BR, D), lambda j: (j, 0)),
        out_shape=jax.ShapeDtypeStruct((N, D), jnp.float32),
    )(acc, den)
    return out
